# baseline (speedup 1.0000x reference)
import jax
import jax.numpy as jnp
from jax import lax
from jax.experimental import pallas as pl
from jax.experimental.pallas import tpu as pltpu

_B, _C, _H, _W = 64, 96, 64, 64


def _body(lab_ref, hp_ref, hprot_ref, l2_ref, kl_ref):
    b = pl.program_id(0)
    x = hp_ref[0]
    y = hprot_ref[0]
    r = lab_ref[b]

    j = jnp.equal(
        lax.broadcasted_iota(jnp.int32, (_W, _W), 0)
        + lax.broadcasted_iota(jnp.int32, (_W, _W), 1),
        _W - 1,
    ).astype(jnp.float32)

    def _mm(a, b_):
        return jax.lax.dot_general(
            a, b_, (((a.ndim - 1,), (0,)), ((), ())),
            preferred_element_type=jnp.float32)

    def _flip_ud(v):
        return jnp.einsum('hu,cuw->chw', j, v,
                          preferred_element_type=jnp.float32)

    xr = lax.switch(
        r,
        (
            lambda v: v,
            lambda v: _mm(jnp.swapaxes(v, 1, 2), j),
            lambda v: _mm(_flip_ud(v), j),
            lambda v: _flip_ud(jnp.swapaxes(v, 1, 2)),
        ),
        x,
    )
    diff = xr - y
    l2 = jnp.sum(diff * diff)
    kl = jnp.sum(xr * jnp.log(xr / jnp.maximum(y, 1e-9)))

    @pl.when(b == 0)
    def _():
        l2_ref[0, 0] = 0.0
        kl_ref[0, 0] = 0.0

    l2_ref[0, 0] += l2
    kl_ref[0, 0] += kl


def kernel(hp, hp_rot, label_rot):
    grid_spec = pltpu.PrefetchScalarGridSpec(
        num_scalar_prefetch=1,
        grid=(_B,),
        in_specs=[
            pl.BlockSpec((1, _C, _H, _W), lambda b, lab: (b, 0, 0, 0)),
            pl.BlockSpec((1, _C, _H, _W), lambda b, lab: (b, 0, 0, 0)),
        ],
        out_specs=[
            pl.BlockSpec(memory_space=pltpu.SMEM, block_shape=(1, 1),
                         index_map=lambda b, lab: (0, 0)),
            pl.BlockSpec(memory_space=pltpu.SMEM, block_shape=(1, 1),
                         index_map=lambda b, lab: (0, 0)),
        ],
    )
    l2, kl = pl.pallas_call(
        _body,
        grid_spec=grid_spec,
        out_shape=[
            jax.ShapeDtypeStruct((1, 1), jnp.float32),
            jax.ShapeDtypeStruct((1, 1), jnp.float32),
        ],
    )(label_rot.astype(jnp.int32), hp, hp_rot)
    kl_s = kl[0, 0] / _B
    l2_s = l2[0, 0] / (_B * _C * _H * _W)
    return kl_s * 0.4 + l2_s * 0.6


# speedup vs baseline: 1.0867x; 1.0867x over previous
import jax
import jax.numpy as jnp
from jax import lax
from jax.experimental import pallas as pl
from jax.experimental.pallas import tpu as pltpu

_B, _C, _H, _W = 64, 96, 64, 64


def _body(lab_ref, hp_ref, hprot_ref, l2_ref, kl_ref):
    b = pl.program_id(0)
    x = hp_ref[0]
    y = hprot_ref[0]
    r = lab_ref[b]

    iota = lax.broadcasted_iota(jnp.int32, (_C, _H, _W), 2)
    rev = (_W - 1) - iota
    idx1 = jnp.where(r == 0, iota, rev)

    xt = jnp.swapaxes(x, 1, 2)
    a = jnp.where((r == 1) | (r == 2), xt, x)
    bb = jnp.take_along_axis(a, idx1, axis=2)
    c = jnp.swapaxes(bb, 1, 2)
    d = jnp.take_along_axis(c, rev, axis=2)
    xr = jnp.where(r == 2, d, jnp.where(r == 3, c, bb))

    diff = xr - y
    l2 = jnp.sum(diff * diff)
    kl = jnp.sum(xr * jnp.log(xr / jnp.maximum(y, 1e-9)))

    @pl.when(b == 0)
    def _():
        l2_ref[0, 0] = 0.0
        kl_ref[0, 0] = 0.0

    l2_ref[0, 0] += l2
    kl_ref[0, 0] += kl


def kernel(hp, hp_rot, label_rot):
    grid_spec = pltpu.PrefetchScalarGridSpec(
        num_scalar_prefetch=1,
        grid=(_B,),
        in_specs=[
            pl.BlockSpec((1, _C, _H, _W), lambda b, lab: (b, 0, 0, 0)),
            pl.BlockSpec((1, _C, _H, _W), lambda b, lab: (b, 0, 0, 0)),
        ],
        out_specs=[
            pl.BlockSpec(memory_space=pltpu.SMEM, block_shape=(1, 1),
                         index_map=lambda b, lab: (0, 0)),
            pl.BlockSpec(memory_space=pltpu.SMEM, block_shape=(1, 1),
                         index_map=lambda b, lab: (0, 0)),
        ],
    )
    l2, kl = pl.pallas_call(
        _body,
        grid_spec=grid_spec,
        out_shape=[
            jax.ShapeDtypeStruct((1, 1), jnp.float32),
            jax.ShapeDtypeStruct((1, 1), jnp.float32),
        ],
    )(label_rot.astype(jnp.int32), hp, hp_rot)
    kl_s = kl[0, 0] / _B
    l2_s = l2[0, 0] / (_B * _C * _H * _W)
    return kl_s * 0.4 + l2_s * 0.6


# speedup vs baseline: 1.1243x; 1.0345x over previous
import jax
import jax.numpy as jnp
from jax import lax
from jax.experimental import pallas as pl
from jax.experimental.pallas import tpu as pltpu

_B, _C, _H, _W = 64, 96, 64, 64
_BB = 2


def _body(lab_ref, hp_ref, hprot_ref, l2_ref, kl_ref):
    step = pl.program_id(0)

    @pl.when(step == 0)
    def _():
        l2_ref[0, 0] = 0.0
        kl_ref[0, 0] = 0.0

    iota = lax.broadcasted_iota(jnp.int32, (_C, _H, _W), 2)
    rev = (_W - 1) - iota

    for i in range(_BB):
        x = hp_ref[i]
        y = hprot_ref[i]
        r = lab_ref[step * _BB + i]

        idx1 = jnp.where(r == 0, iota, rev)
        idxy = jnp.where(r == 2, rev, iota)

        xt = jnp.swapaxes(x, 1, 2)
        a = jnp.where((r == 1) | (r == 2), xt, x)
        bb = jnp.take_along_axis(a, idx1, axis=2)
        c = jnp.swapaxes(bb, 1, 2)
        xr = jnp.where(r <= 1, bb, c)
        yg = jnp.take_along_axis(y, idxy, axis=2)

        diff = xr - yg
        l2_ref[0, 0] += jnp.sum(diff * diff)
        kl_ref[0, 0] += jnp.sum(xr * jnp.log(xr / jnp.maximum(yg, 1e-9)))


def kernel(hp, hp_rot, label_rot):
    grid_spec = pltpu.PrefetchScalarGridSpec(
        num_scalar_prefetch=1,
        grid=(_B // _BB,),
        in_specs=[
            pl.BlockSpec((_BB, _C, _H, _W), lambda b, lab: (b, 0, 0, 0)),
            pl.BlockSpec((_BB, _C, _H, _W), lambda b, lab: (b, 0, 0, 0)),
        ],
        out_specs=[
            pl.BlockSpec(memory_space=pltpu.SMEM, block_shape=(1, 1),
                         index_map=lambda b, lab: (0, 0)),
            pl.BlockSpec(memory_space=pltpu.SMEM, block_shape=(1, 1),
                         index_map=lambda b, lab: (0, 0)),
        ],
    )
    l2, kl = pl.pallas_call(
        _body,
        grid_spec=grid_spec,
        out_shape=[
            jax.ShapeDtypeStruct((1, 1), jnp.float32),
            jax.ShapeDtypeStruct((1, 1), jnp.float32),
        ],
    )(label_rot.astype(jnp.int32), hp, hp_rot)
    kl_s = kl[0, 0] / _B
    l2_s = l2[0, 0] / (_B * _C * _H * _W)
    return kl_s * 0.4 + l2_s * 0.6
